# Initial kernel scaffold; baseline (speedup 1.0000x reference)
import jax
import jax.numpy as jnp
from jax.experimental import pallas as pl


def kernel(input_ids, fill_value):
    raise NotImplementedError("write your pallas kernel here")


# speedup vs baseline: 5.6611x; 5.6611x over previous
import jax
import jax.numpy as jnp
from jax.experimental import pallas as pl
from jax.experimental.pallas import tpu as pltpu

VOCAB = 1024
ROWS = 512


def _onehot_block(fill_ref, ids_ref, out_ref):
    idx = ids_ref[...] % VOCAB
    iota = jax.lax.broadcasted_iota(jnp.int32, (ROWS, VOCAB), 1)
    mask = iota == idx[:, None]
    out_ref[...] = jnp.where(mask, fill_ref[0], jnp.float32(0.0))


def kernel(input_ids, fill_value):
    bs, seq = input_ids.shape
    n = bs * seq
    ids = input_ids.reshape(n)
    fill = fill_value.reshape(1).astype(jnp.float32)
    out = pl.pallas_call(
        _onehot_block,
        grid=(n // ROWS,),
        in_specs=[
            pl.BlockSpec(memory_space=pltpu.SMEM),
            pl.BlockSpec((ROWS,), lambda i: (i,)),
        ],
        out_specs=pl.BlockSpec((ROWS, VOCAB), lambda i: (i, 0)),
        out_shape=jax.ShapeDtypeStruct((n, VOCAB), jnp.float32),
    )(fill, ids)
    return out.reshape(bs, seq, VOCAB)


# speedup vs baseline: 7.3073x; 1.2908x over previous
import jax
import jax.numpy as jnp
from jax.experimental import pallas as pl
from jax.experimental.pallas import tpu as pltpu

VOCAB = 1024
ROWS = 2048


def _onehot_block(fill_ref, ids_ref, out_ref):
    idx = ids_ref[...] % VOCAB
    iota = jax.lax.broadcasted_iota(jnp.int32, (ROWS, VOCAB), 1)
    mask = iota == idx[:, None]
    out_ref[...] = jnp.where(mask, fill_ref[0], jnp.float32(0.0))


def kernel(input_ids, fill_value):
    bs, seq = input_ids.shape
    n = bs * seq
    ids = input_ids.reshape(n)
    fill = fill_value.reshape(1).astype(jnp.float32)
    out = pl.pallas_call(
        _onehot_block,
        grid=(n // ROWS,),
        in_specs=[
            pl.BlockSpec(memory_space=pltpu.SMEM),
            pl.BlockSpec((ROWS,), lambda i: (i,)),
        ],
        out_specs=pl.BlockSpec((ROWS, VOCAB), lambda i: (i, 0)),
        out_shape=jax.ShapeDtypeStruct((n, VOCAB), jnp.float32),
    )(fill, ids)
    return out.reshape(bs, seq, VOCAB)
